# baseline (speedup 1.0000x reference)
import functools

import jax
import jax.numpy as jnp
from jax import lax
from jax.experimental import pallas as pl
from jax.experimental.pallas import tpu as pltpu
from jax.experimental.pallas import tpu_sc as plsc

N_STATES_P1 = 1024
N_OBVS_P1 = 10000
TAB = N_STATES_P1 * N_OBVS_P1
N_ROWS = 16384
ROW = 200
N_TOTAL = N_ROWS * ROW

_FULL = 9984
_TAILW = N_OBVS_P1 - _FULL
_TAILB = N_STATES_P1 * _FULL

_info = plsc.get_sparse_core_info()
_NC, _NS, _L = _info.num_cores, _info.num_subcores, _info.num_lanes
_NW = _NC * _NS
_ROWS_PER_TILE = N_ROWS // _NW
_CROWS = 16
_CHUNK = _CROWS * ROW
_N_CHUNKS = _ROWS_PER_TILE // _CROWS
_BLKS_PER_TILE = N_STATES_P1 // 8 // _NS

_CS_ALIGNED = tuple(range(0, ROW - _L, _L))
_CS_READ = _CS_ALIGNED + (ROW - _L,)

_mesh = plsc.VectorSubcoreMesh(core_axis_name="c", subcore_axis_name="s")


@functools.partial(
    pl.kernel,
    out_type=(
        jax.ShapeDtypeStruct((N_TOTAL,), jnp.float32),
        jax.ShapeDtypeStruct((_NC * TAB,), jnp.float32),
    ),
    mesh=_mesh,
    scratch_types=[
        pltpu.VMEM((8, _FULL), jnp.float32),
        pltpu.VMEM((8, _TAILW), jnp.float32),
        pltpu.VMEM((8 * _TAILW,), jnp.float32),
        pltpu.VMEM((_CROWS, ROW), jnp.int32),
        pltpu.VMEM((_CROWS, ROW), jnp.int32),
        pltpu.VMEM((_CROWS, ROW), jnp.int32),
        pltpu.VMEM((_CROWS, ROW), jnp.int32),
        pltpu.VMEM((_CHUNK,), jnp.int32),
        pltpu.VMEM((_CHUNK,), jnp.int32),
        pltpu.VMEM((_CHUNK,), jnp.float32),
        pltpu.VMEM((_CHUNK,), jnp.float32),
        pltpu.SemaphoreType.DMA,
        pltpu.SemaphoreType.DMA,
        pltpu.SemaphoreType.DMA,
        pltpu.SemaphoreType.DMA,
        pltpu.SemaphoreType.DMA,
        pltpu.SemaphoreType.DMA,
        pltpu.SemaphoreType.DMA,
        pltpu.SemaphoreType.DMA,
    ],
)
def _sc_emission(table_hbm, state_hbm, obs_hbm, out_hbm, flat_hbm,
                 fbuf, tbuf, tstage, s0, s1, o0, o1, i0, i1, g0, g1,
                 in_sem0, in_sem1, g_sem0, g_sem1, wb_sem0, wb_sem1,
                 fl_sem, fs_sem):
    s = (s0, s1)
    o = (o0, o1)
    idx = (i0, i1)
    g = (g0, g1)
    in_sem = (in_sem0, in_sem1)
    g_sem = (g_sem0, g_sem1)
    wb_sem = (wb_sem0, wb_sem1)
    sc = lax.axis_index("c")
    tid = lax.axis_index("s")
    wid = tid * _NC + sc
    row_base = wid * _ROWS_PER_TILE
    sc_off = sc * TAB

    def start_in(p, c):
        rw = row_base + c * _CROWS
        pltpu.async_copy(state_hbm.at[pl.ds(rw, _CROWS), :], s[p], in_sem[p])
        pltpu.async_copy(obs_hbm.at[pl.ds(rw, _CROWS), :], o[p], in_sem[p])

    def wait_in(p):
        pltpu.make_async_copy(
            state_hbm.at[pl.ds(0, _CROWS), :], s[p], in_sem[p]).wait()
        pltpu.make_async_copy(
            obs_hbm.at[pl.ds(0, _CROWS), :], o[p], in_sem[p]).wait()

    def start_gather(p):
        pltpu.async_copy(flat_hbm.at[idx[p]], g[p], g_sem[p])

    def wait_gather(p):
        pltpu.make_async_copy(flat_hbm.at[idx[p]], g[p], g_sem[p]).wait()

    def start_wb(p, c):
        off = (row_base + c * _CROWS) * ROW
        pltpu.async_copy(g[p], out_hbm.at[pl.ds(off, _CHUNK)], wb_sem[p])

    def wait_wb(p):
        pltpu.make_async_copy(
            g[p], out_hbm.at[pl.ds(0, _CHUNK)], wb_sem[p]).wait()

    def idx_pass(p):
        sb, ob, ib = s[p], o[p], idx[p]

        def body(r, carry):
            for cs in _CS_READ:
                sv = sb[r, pl.ds(cs, _L)]
                ov = ob[r, pl.ds(cs, _L)]
                off_full = sv * _FULL + ov
                off_tail = _TAILB + (sv << 4) + (ov - _FULL)
                ib[pl.ds(r * ROW + cs, _L)] = (
                    jnp.where(ov < _FULL, off_full, off_tail) + sc_off)
            return carry

        lax.fori_loop(0, _CROWS, body, 0)

    start_in(0, 0)
    start_in(1, 1)

    for j in range(_BLKS_PER_TILE):
        blk = tid * _BLKS_PER_TILE + j
        hf = pltpu.async_copy(
            table_hbm.at[pl.ds(blk * 8, 8), pl.ds(0, _FULL)], fbuf, fl_sem)
        ht = pltpu.async_copy(
            table_hbm.at[pl.ds(blk * 8, 8), pl.ds(_FULL, _TAILW)], tbuf, fl_sem)
        hf.wait()
        ht.wait()
        hs = []
        for r in range(8):
            hs.append(pltpu.async_copy(
                fbuf.at[r, :],
                flat_hbm.at[pl.ds(sc_off + (blk * 8 + r) * _FULL, _FULL)],
                fs_sem))
            tstage[pl.ds(r * _TAILW, _TAILW)] = tbuf[r, pl.ds(0, _TAILW)]
        hs.append(pltpu.async_copy(
            tstage,
            flat_hbm.at[pl.ds(sc_off + _TAILB + blk * 8 * _TAILW, 8 * _TAILW)],
            fs_sem))
        for h in hs:
            h.wait()
    plsc.subcore_barrier()

    def pair(i, carry):
        c0 = 2 * i
        c1 = c0 + 1
        wait_in(0)
        idx_pass(0)

        @pl.when(c0 + 2 < _N_CHUNKS)
        def _():
            start_in(0, c0 + 2)

        @pl.when(i > 0)
        def _():
            wait_gather(1)
            start_wb(1, c0 - 1)
            wait_wb(0)

        start_gather(0)
        wait_in(1)
        idx_pass(1)

        @pl.when(c1 + 2 < _N_CHUNKS)
        def _():
            start_in(1, c1 + 2)

        wait_gather(0)
        start_wb(0, c0)

        @pl.when(i > 0)
        def _():
            wait_wb(1)

        start_gather(1)
        return carry

    lax.fori_loop(0, _N_CHUNKS // 2, pair, 0)

    last = _N_CHUNKS - 1
    wait_gather(1)
    start_wb(1, last)
    wait_wb(0)
    wait_wb(1)


def kernel(state, obs, log_em):
    out, _ = _sc_emission(log_em, state, obs)
    return out.reshape(state.shape)


# speedup vs baseline: 1.0078x; 1.0078x over previous
import functools

import jax
import jax.numpy as jnp
from jax import lax
from jax.experimental import pallas as pl
from jax.experimental.pallas import tpu as pltpu
from jax.experimental.pallas import tpu_sc as plsc

N_STATES_P1 = 1024
N_OBVS_P1 = 10000
TAB = N_STATES_P1 * N_OBVS_P1
N_ROWS = 16384
ROW = 200
N_TOTAL = N_ROWS * ROW

_FULL = 9984
_TAILW = N_OBVS_P1 - _FULL
_TAILB = N_STATES_P1 * _FULL

_info = plsc.get_sparse_core_info()
_NC, _NS, _L = _info.num_cores, _info.num_subcores, _info.num_lanes
_NW = _NC * _NS
_ROWS_PER_TILE = N_ROWS // _NW
_CROWS = 32
_CHUNK = _CROWS * ROW
_N_CHUNKS = _ROWS_PER_TILE // _CROWS
_BLKS_PER_TILE = N_STATES_P1 // 8 // _NS
_PIECE = 3328
_NPIECE = _FULL // _PIECE
_N_PIECES = _BLKS_PER_TILE * _NPIECE

_CS_ALIGNED = tuple(range(0, ROW - _L, _L))
_CS_READ = _CS_ALIGNED + (ROW - _L,)

_mesh = plsc.VectorSubcoreMesh(core_axis_name="c", subcore_axis_name="s")


@functools.partial(
    pl.kernel,
    out_type=jax.ShapeDtypeStruct((N_TOTAL,), jnp.float32),
    mesh=_mesh,
    scratch_types=[
        pltpu.HBM((_NC * TAB,), jnp.float32),
        pltpu.VMEM((8, _PIECE), jnp.float32),
        pltpu.VMEM((8, _PIECE), jnp.float32),
        pltpu.VMEM((8, _TAILW), jnp.float32),
        pltpu.VMEM((8 * _TAILW,), jnp.float32),
        pltpu.VMEM((_CROWS, ROW), jnp.int32),
        pltpu.VMEM((_CROWS, ROW), jnp.int32),
        pltpu.VMEM((_CROWS, ROW), jnp.int32),
        pltpu.VMEM((_CROWS, ROW), jnp.int32),
        pltpu.VMEM((_CHUNK,), jnp.int32),
        pltpu.VMEM((_CHUNK,), jnp.int32),
        pltpu.VMEM((_CHUNK,), jnp.float32),
        pltpu.VMEM((_CHUNK,), jnp.float32),
        pltpu.SemaphoreType.DMA,
        pltpu.SemaphoreType.DMA,
        pltpu.SemaphoreType.DMA,
        pltpu.SemaphoreType.DMA,
        pltpu.SemaphoreType.DMA,
        pltpu.SemaphoreType.DMA,
        pltpu.SemaphoreType.DMA,
        pltpu.SemaphoreType.DMA,
    ],
)
def _sc_emission(table_hbm, state_hbm, obs_hbm, out_hbm, flat_hbm,
                 fb0, fb1, tbuf, tstage, s0, s1, o0, o1, i0, i1, g0, g1,
                 in_sem0, in_sem1, g_sem0, g_sem1, wb_sem0, wb_sem1,
                 fl_sem, fs_sem):
    fb = (fb0, fb1)
    s = (s0, s1)
    o = (o0, o1)
    idx = (i0, i1)
    g = (g0, g1)
    in_sem = (in_sem0, in_sem1)
    g_sem = (g_sem0, g_sem1)
    wb_sem = (wb_sem0, wb_sem1)
    sc = lax.axis_index("c")
    tid = lax.axis_index("s")
    wid = tid * _NC + sc
    row_base = wid * _ROWS_PER_TILE
    sc_off = sc * TAB

    def start_in(p, c):
        rw = row_base + c * _CROWS
        pltpu.async_copy(state_hbm.at[pl.ds(rw, _CROWS), :], s[p], in_sem[p])
        pltpu.async_copy(obs_hbm.at[pl.ds(rw, _CROWS), :], o[p], in_sem[p])

    def wait_in(p):
        pltpu.make_async_copy(
            state_hbm.at[pl.ds(0, _CROWS), :], s[p], in_sem[p]).wait()
        pltpu.make_async_copy(
            obs_hbm.at[pl.ds(0, _CROWS), :], o[p], in_sem[p]).wait()

    def start_gather(p):
        pltpu.async_copy(flat_hbm.at[idx[p]], g[p], g_sem[p])

    def wait_gather(p):
        pltpu.make_async_copy(flat_hbm.at[idx[p]], g[p], g_sem[p]).wait()

    def start_wb(p, c):
        off = (row_base + c * _CROWS) * ROW
        pltpu.async_copy(g[p], out_hbm.at[pl.ds(off, _CHUNK)], wb_sem[p])

    def wait_wb(p):
        pltpu.make_async_copy(
            g[p], out_hbm.at[pl.ds(0, _CHUNK)], wb_sem[p]).wait()

    def idx_pass(p):
        sb, ob, ib = s[p], o[p], idx[p]

        def body(r, carry):
            for cs in _CS_READ:
                sv = sb[r, pl.ds(cs, _L)]
                ov = ob[r, pl.ds(cs, _L)]
                off_full = sv * _FULL + ov
                off_tail = _TAILB + (sv << 4) + (ov - _FULL)
                ib[pl.ds(r * ROW + cs, _L)] = (
                    jnp.where(ov < _FULL, off_full, off_tail) + sc_off)
            return carry

        lax.fori_loop(0, _CROWS, body, 0)

    start_in(0, 0)
    start_in(1, 1)

    for j in range(_BLKS_PER_TILE):
        blk = tid * _BLKS_PER_TILE + j
        pltpu.async_copy(
            table_hbm.at[pl.ds(blk * 8, 8), pl.ds(_FULL, _TAILW)], tbuf,
            fl_sem).wait()
        for r in range(8):
            tstage[pl.ds(r * _TAILW, _TAILW)] = tbuf[r, pl.ds(0, _TAILW)]
        pltpu.async_copy(
            tstage,
            flat_hbm.at[pl.ds(sc_off + _TAILB + blk * 8 * _TAILW, 8 * _TAILW)],
            fs_sem).wait()

    def start_pin(q, b):
        blk = tid * _BLKS_PER_TILE + q // _NPIECE
        k = q % _NPIECE
        return pltpu.async_copy(
            table_hbm.at[pl.ds(blk * 8, 8), pl.ds(k * _PIECE, _PIECE)],
            fb[b], fl_sem)

    fl_h = {0: start_pin(0, 0)}
    fs_h = {}
    for q in range(_N_PIECES):
        b = q & 1
        fl_h.pop(q).wait()
        if q >= 1:
            for h in fs_h.pop(q - 1):
                h.wait()
        if q + 1 < _N_PIECES:
            fl_h[q + 1] = start_pin(q + 1, 1 - b)
        blk = tid * _BLKS_PER_TILE + q // _NPIECE
        k = q % _NPIECE
        hs = []
        for r in range(8):
            hs.append(pltpu.async_copy(
                fb[b].at[r, :],
                flat_hbm.at[pl.ds(
                    sc_off + (blk * 8 + r) * _FULL + k * _PIECE, _PIECE)],
                fs_sem))
        fs_h[q] = hs
    for h in fs_h.pop(_N_PIECES - 1):
        h.wait()
    plsc.subcore_barrier()

    def pair(i, carry):
        c0 = 2 * i
        c1 = c0 + 1
        wait_in(0)
        idx_pass(0)

        @pl.when(c0 + 2 < _N_CHUNKS)
        def _():
            start_in(0, c0 + 2)

        @pl.when(i > 0)
        def _():
            wait_gather(1)
            start_wb(1, c0 - 1)
            wait_wb(0)

        start_gather(0)
        wait_in(1)
        idx_pass(1)

        @pl.when(c1 + 2 < _N_CHUNKS)
        def _():
            start_in(1, c1 + 2)

        wait_gather(0)
        start_wb(0, c0)

        @pl.when(i > 0)
        def _():
            wait_wb(1)

        start_gather(1)
        return carry

    lax.fori_loop(0, _N_CHUNKS // 2, pair, 0)

    last = _N_CHUNKS - 1
    wait_gather(1)
    start_wb(1, last)
    wait_wb(0)
    wait_wb(1)


def kernel(state, obs, log_em):
    out = _sc_emission(log_em, state, obs)
    return out.reshape(state.shape)


# speedup vs baseline: 1.1680x; 1.1590x over previous
import functools

import jax
import jax.numpy as jnp
from jax import lax
from jax.experimental import pallas as pl
from jax.experimental.pallas import tpu as pltpu
from jax.experimental.pallas import tpu_sc as plsc

N_OBVS_P1 = 10000
N_ROWS = 16384
ROW = 200
N_TOTAL = N_ROWS * ROW

_info = plsc.get_sparse_core_info()
_NC, _NS, _L = _info.num_cores, _info.num_subcores, _info.num_lanes
_NW = _NC * _NS
_ROWS_PER_TILE = N_ROWS // _NW
_CROWS = 32
_CHUNK = _CROWS * ROW
_N_CHUNKS = _ROWS_PER_TILE // _CROWS

_CS_READ = tuple(range(0, ROW - _L, _L)) + (ROW - _L,)

_mesh = plsc.VectorSubcoreMesh(core_axis_name="c", subcore_axis_name="s")


@functools.partial(
    pl.kernel,
    out_type=jax.ShapeDtypeStruct((N_TOTAL,), jnp.float32),
    mesh=_mesh,
    scratch_types=[
        pltpu.VMEM((_CROWS, ROW), jnp.int32),
        pltpu.VMEM((_CROWS, ROW), jnp.int32),
        pltpu.VMEM((_CROWS, ROW), jnp.int32),
        pltpu.VMEM((_CROWS, ROW), jnp.int32),
        pltpu.VMEM((_CHUNK,), jnp.int32),
        pltpu.VMEM((_CHUNK,), jnp.int32),
        pltpu.VMEM((_CHUNK,), jnp.float32),
        pltpu.VMEM((_CHUNK,), jnp.float32),
        pltpu.SemaphoreType.DMA,
        pltpu.SemaphoreType.DMA,
        pltpu.SemaphoreType.DMA,
        pltpu.SemaphoreType.DMA,
        pltpu.SemaphoreType.DMA,
        pltpu.SemaphoreType.DMA,
    ],
)
def _sc_emission(table_hbm, state_hbm, obs_hbm, out_hbm,
                 s0, s1, o0, o1, i0, i1, g0, g1,
                 in_sem0, in_sem1, g_sem0, g_sem1, wb_sem0, wb_sem1):
    s = (s0, s1)
    o = (o0, o1)
    idx = (i0, i1)
    g = (g0, g1)
    in_sem = (in_sem0, in_sem1)
    g_sem = (g_sem0, g_sem1)
    wb_sem = (wb_sem0, wb_sem1)
    sc = lax.axis_index("c")
    tid = lax.axis_index("s")
    wid = tid * _NC + sc
    row_base = wid * _ROWS_PER_TILE

    def start_in(p, c):
        rw = row_base + c * _CROWS
        pltpu.async_copy(state_hbm.at[pl.ds(rw, _CROWS), :], s[p], in_sem[p])
        pltpu.async_copy(obs_hbm.at[pl.ds(rw, _CROWS), :], o[p], in_sem[p])

    def wait_in(p):
        pltpu.make_async_copy(
            state_hbm.at[pl.ds(0, _CROWS), :], s[p], in_sem[p]).wait()
        pltpu.make_async_copy(
            obs_hbm.at[pl.ds(0, _CROWS), :], o[p], in_sem[p]).wait()

    def start_gather(p):
        pltpu.async_copy(table_hbm.at[idx[p]], g[p], g_sem[p])

    def wait_gather(p):
        pltpu.make_async_copy(table_hbm.at[idx[p]], g[p], g_sem[p]).wait()

    def start_wb(p, c):
        off = (row_base + c * _CROWS) * ROW
        pltpu.async_copy(g[p], out_hbm.at[pl.ds(off, _CHUNK)], wb_sem[p])

    def wait_wb(p):
        pltpu.make_async_copy(
            g[p], out_hbm.at[pl.ds(0, _CHUNK)], wb_sem[p]).wait()

    def idx_pass(p):
        sb, ob, ib = s[p], o[p], idx[p]

        def body(r, carry):
            for cs in _CS_READ:
                sv = sb[r, pl.ds(cs, _L)]
                ov = ob[r, pl.ds(cs, _L)]
                ib[pl.ds(r * ROW + cs, _L)] = sv * N_OBVS_P1 + ov
            return carry

        lax.fori_loop(0, _CROWS, body, 0)

    start_in(0, 0)
    start_in(1, 1)

    def pair(i, carry):
        c0 = 2 * i
        c1 = c0 + 1
        wait_in(0)
        idx_pass(0)

        @pl.when(c0 + 2 < _N_CHUNKS)
        def _():
            start_in(0, c0 + 2)

        @pl.when(i > 0)
        def _():
            wait_gather(1)
            start_wb(1, c0 - 1)
            wait_wb(0)

        start_gather(0)
        wait_in(1)
        idx_pass(1)

        @pl.when(c1 + 2 < _N_CHUNKS)
        def _():
            start_in(1, c1 + 2)

        wait_gather(0)
        start_wb(0, c0)

        @pl.when(i > 0)
        def _():
            wait_wb(1)

        start_gather(1)
        return carry

    lax.fori_loop(0, _N_CHUNKS // 2, pair, 0)

    last = _N_CHUNKS - 1
    wait_gather(1)
    start_wb(1, last)
    wait_wb(0)
    wait_wb(1)


def kernel(state, obs, log_em):
    out = _sc_emission(log_em.reshape(-1), state, obs)
    return out.reshape(state.shape)


# speedup vs baseline: 1.1724x; 1.0038x over previous
import functools

import jax
import jax.numpy as jnp
from jax import lax
from jax.experimental import pallas as pl
from jax.experimental.pallas import tpu as pltpu
from jax.experimental.pallas import tpu_sc as plsc

N_OBVS_P1 = 10000
N_ROWS = 16384
ROW = 200
N_TOTAL = N_ROWS * ROW

_info = plsc.get_sparse_core_info()
_NC, _NS, _L = _info.num_cores, _info.num_subcores, _info.num_lanes
_NW = _NC * _NS
_ROWS_PER_TILE = N_ROWS // _NW
_CROWS = 32
_CHUNK = _CROWS * ROW
_N_CHUNKS = _ROWS_PER_TILE // _CROWS

_CS_READ = tuple(range(0, ROW - _L, _L)) + (ROW - _L,)

_mesh = plsc.VectorSubcoreMesh(core_axis_name="c", subcore_axis_name="s")


@functools.partial(
    pl.kernel,
    out_type=jax.ShapeDtypeStruct((N_TOTAL,), jnp.float32),
    mesh=_mesh,
    scratch_types=[
        pltpu.VMEM((_CROWS, ROW), jnp.int32),
        pltpu.VMEM((_CROWS, ROW), jnp.int32),
        pltpu.VMEM((_CROWS, ROW), jnp.int32),
        pltpu.VMEM((_CROWS, ROW), jnp.int32),
        pltpu.VMEM((_CHUNK,), jnp.int32),
        pltpu.VMEM((_CHUNK,), jnp.int32),
        pltpu.VMEM((_CHUNK,), jnp.float32),
        pltpu.VMEM((_CHUNK,), jnp.float32),
        pltpu.SemaphoreType.DMA,
        pltpu.SemaphoreType.DMA,
        pltpu.SemaphoreType.DMA,
        pltpu.SemaphoreType.DMA,
        pltpu.SemaphoreType.DMA,
        pltpu.SemaphoreType.DMA,
    ],
)
def _sc_emission(table_hbm, state_hbm, obs_hbm, out_hbm,
                 s0, s1, o0, o1, i0, i1, g0, g1,
                 in_sem0, in_sem1, g_sem0, g_sem1, wb_sem0, wb_sem1):
    s = (s0, s1)
    o = (o0, o1)
    idx = (i0, i1)
    g = (g0, g1)
    in_sem = (in_sem0, in_sem1)
    g_sem = (g_sem0, g_sem1)
    wb_sem = (wb_sem0, wb_sem1)
    sc = lax.axis_index("c")
    tid = lax.axis_index("s")
    wid = tid * _NC + sc
    row_base = wid * _ROWS_PER_TILE

    def start_in(p, c):
        rw = row_base + c * _CROWS
        pltpu.async_copy(state_hbm.at[pl.ds(rw, _CROWS), :], s[p], in_sem[p])
        pltpu.async_copy(obs_hbm.at[pl.ds(rw, _CROWS), :], o[p], in_sem[p])

    def wait_in(p):
        pltpu.make_async_copy(
            state_hbm.at[pl.ds(0, _CROWS), :], s[p], in_sem[p]).wait()
        pltpu.make_async_copy(
            obs_hbm.at[pl.ds(0, _CROWS), :], o[p], in_sem[p]).wait()

    def start_gather(p):
        pltpu.async_copy(table_hbm.at[idx[p]], g[p], g_sem[p])

    def wait_gather(p):
        pltpu.make_async_copy(table_hbm.at[idx[p]], g[p], g_sem[p]).wait()

    def start_wb(p, c):
        off = (row_base + c * _CROWS) * ROW
        pltpu.async_copy(g[p], out_hbm.at[pl.ds(off, _CHUNK)], wb_sem[p])

    def wait_wb(p):
        pltpu.make_async_copy(
            g[p], out_hbm.at[pl.ds(0, _CHUNK)], wb_sem[p]).wait()

    def idx_pass(p):
        sb, ob, ib = s[p], o[p], idx[p]

        @plsc.parallel_loop(0, _CROWS, step=1, unroll=2)
        def _(r):
            for cs in _CS_READ:
                sv = sb[r, pl.ds(cs, _L)]
                ov = ob[r, pl.ds(cs, _L)]
                ib[pl.ds(r * ROW + cs, _L)] = sv * N_OBVS_P1 + ov

    start_in(0, 0)
    start_in(1, 1)

    def pair(i, carry):
        c0 = 2 * i
        c1 = c0 + 1
        wait_in(0)
        idx_pass(0)

        @pl.when(c0 + 2 < _N_CHUNKS)
        def _():
            start_in(0, c0 + 2)

        @pl.when(i > 0)
        def _():
            wait_gather(1)
            start_wb(1, c0 - 1)
            wait_wb(0)

        start_gather(0)
        wait_in(1)
        idx_pass(1)

        @pl.when(c1 + 2 < _N_CHUNKS)
        def _():
            start_in(1, c1 + 2)

        wait_gather(0)
        start_wb(0, c0)

        @pl.when(i > 0)
        def _():
            wait_wb(1)

        start_gather(1)
        return carry

    lax.fori_loop(0, _N_CHUNKS // 2, pair, 0)

    last = _N_CHUNKS - 1
    wait_gather(1)
    start_wb(1, last)
    wait_wb(0)
    wait_wb(1)


def kernel(state, obs, log_em):
    out = _sc_emission(log_em.reshape(-1), state, obs)
    return out.reshape(state.shape)


# speedup vs baseline: 1.1830x; 1.0091x over previous
import functools

import jax
import jax.numpy as jnp
from jax import lax
from jax.experimental import pallas as pl
from jax.experimental.pallas import tpu as pltpu
from jax.experimental.pallas import tpu_sc as plsc

N_OBVS_P1 = 10000
N_ROWS = 16384
ROW = 200
N_TOTAL = N_ROWS * ROW

_info = plsc.get_sparse_core_info()
_NC, _NS, _L = _info.num_cores, _info.num_subcores, _info.num_lanes
_NW = _NC * _NS
_ROWS_PER_TILE = N_ROWS // _NW
_CROWS = 64
_CHUNK = _CROWS * ROW
_N_CHUNKS = _ROWS_PER_TILE // _CROWS

_CS_READ = tuple(range(0, ROW - _L, _L)) + (ROW - _L,)

_mesh = plsc.VectorSubcoreMesh(core_axis_name="c", subcore_axis_name="s")


@functools.partial(
    pl.kernel,
    out_type=jax.ShapeDtypeStruct((N_TOTAL,), jnp.float32),
    mesh=_mesh,
    scratch_types=[
        pltpu.VMEM((_CROWS, ROW), jnp.int32),
        pltpu.VMEM((_CROWS, ROW), jnp.int32),
        pltpu.VMEM((_CROWS, ROW), jnp.int32),
        pltpu.VMEM((_CROWS, ROW), jnp.int32),
        pltpu.VMEM((_CHUNK,), jnp.int32),
        pltpu.VMEM((_CHUNK,), jnp.int32),
        pltpu.VMEM((_CHUNK,), jnp.float32),
        pltpu.VMEM((_CHUNK,), jnp.float32),
        pltpu.SemaphoreType.DMA,
        pltpu.SemaphoreType.DMA,
        pltpu.SemaphoreType.DMA,
        pltpu.SemaphoreType.DMA,
        pltpu.SemaphoreType.DMA,
        pltpu.SemaphoreType.DMA,
    ],
)
def _sc_emission(table_hbm, state_hbm, obs_hbm, out_hbm,
                 s0, s1, o0, o1, i0, i1, g0, g1,
                 in_sem0, in_sem1, g_sem0, g_sem1, wb_sem0, wb_sem1):
    s = (s0, s1)
    o = (o0, o1)
    idx = (i0, i1)
    g = (g0, g1)
    in_sem = (in_sem0, in_sem1)
    g_sem = (g_sem0, g_sem1)
    wb_sem = (wb_sem0, wb_sem1)
    sc = lax.axis_index("c")
    tid = lax.axis_index("s")
    wid = tid * _NC + sc
    row_base = wid * _ROWS_PER_TILE

    def start_in(p, c):
        rw = row_base + c * _CROWS
        pltpu.async_copy(state_hbm.at[pl.ds(rw, _CROWS), :], s[p], in_sem[p])
        pltpu.async_copy(obs_hbm.at[pl.ds(rw, _CROWS), :], o[p], in_sem[p])

    def wait_in(p):
        pltpu.make_async_copy(
            state_hbm.at[pl.ds(0, _CROWS), :], s[p], in_sem[p]).wait()
        pltpu.make_async_copy(
            obs_hbm.at[pl.ds(0, _CROWS), :], o[p], in_sem[p]).wait()

    def start_gather(p):
        pltpu.async_copy(table_hbm.at[idx[p]], g[p], g_sem[p])

    def wait_gather(p):
        pltpu.make_async_copy(table_hbm.at[idx[p]], g[p], g_sem[p]).wait()

    def start_wb(p, c):
        off = (row_base + c * _CROWS) * ROW
        pltpu.async_copy(g[p], out_hbm.at[pl.ds(off, _CHUNK)], wb_sem[p])

    def wait_wb(p):
        pltpu.make_async_copy(
            g[p], out_hbm.at[pl.ds(0, _CHUNK)], wb_sem[p]).wait()

    def idx_pass(p):
        sb, ob, ib = s[p], o[p], idx[p]

        @plsc.parallel_loop(0, _CROWS, step=1, unroll=2)
        def _(r):
            for cs in _CS_READ:
                sv = sb[r, pl.ds(cs, _L)]
                ov = ob[r, pl.ds(cs, _L)]
                ib[pl.ds(r * ROW + cs, _L)] = sv * N_OBVS_P1 + ov

    start_in(0, 0)
    start_in(1, 1)

    def pair(i, carry):
        c0 = 2 * i
        c1 = c0 + 1
        wait_in(0)
        idx_pass(0)

        @pl.when(c0 + 2 < _N_CHUNKS)
        def _():
            start_in(0, c0 + 2)

        @pl.when(i > 0)
        def _():
            wait_gather(1)
            start_wb(1, c0 - 1)
            wait_wb(0)

        start_gather(0)
        wait_in(1)
        idx_pass(1)

        @pl.when(c1 + 2 < _N_CHUNKS)
        def _():
            start_in(1, c1 + 2)

        wait_gather(0)
        start_wb(0, c0)

        @pl.when(i > 0)
        def _():
            wait_wb(1)

        start_gather(1)
        return carry

    lax.fori_loop(0, _N_CHUNKS // 2, pair, 0)

    last = _N_CHUNKS - 1
    wait_gather(1)
    start_wb(1, last)
    wait_wb(0)
    wait_wb(1)


def kernel(state, obs, log_em):
    out = _sc_emission(log_em.reshape(-1), state, obs)
    return out.reshape(state.shape)


# speedup vs baseline: 1.1836x; 1.0005x over previous
import functools

import jax
import jax.numpy as jnp
from jax import lax
from jax.experimental import pallas as pl
from jax.experimental.pallas import tpu as pltpu
from jax.experimental.pallas import tpu_sc as plsc

N_OBVS_P1 = 10000
N_ROWS = 16384
ROW = 200
N_TOTAL = N_ROWS * ROW

_info = plsc.get_sparse_core_info()
_NC, _NS, _L = _info.num_cores, _info.num_subcores, _info.num_lanes
_NW = _NC * _NS
_ROWS_PER_TILE = N_ROWS // _NW
_CROWS = 64
_CHUNK = _CROWS * ROW
_N_CHUNKS = _ROWS_PER_TILE // _CROWS

_CS_READ = tuple(range(0, ROW - _L, _L)) + (ROW - _L,)

_mesh = plsc.VectorSubcoreMesh(core_axis_name="c", subcore_axis_name="s")


@functools.partial(
    pl.kernel,
    out_type=jax.ShapeDtypeStruct((N_TOTAL,), jnp.float32),
    mesh=_mesh,
    scratch_types=[
        pltpu.VMEM((_CROWS, ROW), jnp.int32),
        pltpu.VMEM((_CROWS, ROW), jnp.int32),
        pltpu.VMEM((_CROWS, ROW), jnp.int32),
        pltpu.VMEM((_CROWS, ROW), jnp.int32),
        pltpu.VMEM((_CHUNK,), jnp.int32),
        pltpu.VMEM((_CHUNK,), jnp.int32),
        pltpu.VMEM((_CHUNK,), jnp.float32),
        pltpu.VMEM((_CHUNK,), jnp.float32),
        pltpu.SemaphoreType.DMA,
        pltpu.SemaphoreType.DMA,
        pltpu.SemaphoreType.DMA,
        pltpu.SemaphoreType.DMA,
        pltpu.SemaphoreType.DMA,
        pltpu.SemaphoreType.DMA,
    ],
)
def _sc_emission(table_hbm, state_hbm, obs_hbm, out_hbm,
                 s0, s1, o0, o1, i0, i1, g0, g1,
                 in_sem0, in_sem1, g_sem0, g_sem1, wb_sem0, wb_sem1):
    s = (s0, s1)
    o = (o0, o1)
    idx = (i0, i1)
    g = (g0, g1)
    in_sem = (in_sem0, in_sem1)
    g_sem = (g_sem0, g_sem1)
    wb_sem = (wb_sem0, wb_sem1)
    sc = lax.axis_index("c")
    tid = lax.axis_index("s")
    wid = tid * _NC + sc
    row_base = wid * _ROWS_PER_TILE

    def start_in(p, c):
        rw = row_base + c * _CROWS
        pltpu.async_copy(state_hbm.at[pl.ds(rw, _CROWS), :], s[p], in_sem[p])
        pltpu.async_copy(obs_hbm.at[pl.ds(rw, _CROWS), :], o[p], in_sem[p])

    def wait_in(p):
        pltpu.make_async_copy(
            state_hbm.at[pl.ds(0, _CROWS), :], s[p], in_sem[p]).wait()
        pltpu.make_async_copy(
            obs_hbm.at[pl.ds(0, _CROWS), :], o[p], in_sem[p]).wait()

    def start_gather(p):
        pltpu.async_copy(table_hbm.at[idx[p]], g[p], g_sem[p])

    def wait_gather(p):
        pltpu.make_async_copy(table_hbm.at[idx[p]], g[p], g_sem[p]).wait()

    def start_wb(p, c):
        off = (row_base + c * _CROWS) * ROW
        pltpu.async_copy(g[p], out_hbm.at[pl.ds(off, _CHUNK)], wb_sem[p])

    def wait_wb(p):
        pltpu.make_async_copy(
            g[p], out_hbm.at[pl.ds(0, _CHUNK)], wb_sem[p]).wait()

    def idx_pass(p):
        sb, ob, ib = s[p], o[p], idx[p]

        @plsc.parallel_loop(0, _CROWS, step=1, unroll=2)
        def _(r):
            for cs in _CS_READ:
                sv = sb[r, pl.ds(cs, _L)]
                ov = ob[r, pl.ds(cs, _L)]
                ib[pl.ds(r * ROW + cs, _L)] = sv * N_OBVS_P1 + ov

    start_in(0, 0)
    start_in(1, 1)

    def pair(i, carry):
        c0 = 2 * i
        c1 = c0 + 1
        wait_in(0)
        idx_pass(0)

        @pl.when(c0 + 2 < _N_CHUNKS)
        def _():
            start_in(0, c0 + 2)

        @pl.when(i > 0)
        def _():
            wait_gather(1)
            start_wb(1, c0 - 1)
            wait_wb(0)

        start_gather(0)
        wait_in(1)
        idx_pass(1)

        @pl.when(c1 + 2 < _N_CHUNKS)
        def _():
            start_in(1, c1 + 2)

        wait_gather(0)
        start_wb(0, c0)

        @pl.when(i > 0)
        def _():
            wait_wb(1)

        start_gather(1)
        return carry

    lax.fori_loop(0, _N_CHUNKS // 2, pair, 0)

    last = _N_CHUNKS - 1
    wait_gather(1)
    start_wb(1, last)
    wait_wb(0)
    wait_wb(1)


def kernel(state, obs, log_em):
    out = _sc_emission((log_em + 0.0).reshape(-1), state, obs)
    return out.reshape(state.shape)
